# baseline (speedup 1.0000x reference)
import functools

import jax
import jax.numpy as jnp
from jax import lax
from jax.experimental import pallas as pl
from jax.experimental.pallas import tpu as pltpu
from jax.experimental.pallas import tpu_sc as plsc

_NC = 2
_NS = 16
_NW = _NC * _NS
_L = 16
_BC = 512
_ITEM = 32
_D = 64


def _tec_body(ids_hbm, rat_hbm, pe_hbm, posb_hbm, re8_hbm,
              out_hbm, mask_hbm,
              ids_v0, rat_v0, pe_v0, posb_v0, out_v0, mask_v0,
              ids_v1, rat_v1, pe_v1, posb_v1, out_v1, mask_v1,
              re_v, in_sem, out_sem):
    bw = pe_hbm.shape[1]
    n_rows = posb_hbm.shape[0] // (_D * _L)
    nb = bw // _BC
    units = n_rows * nb // _NW
    wid = lax.axis_index("s") * _NC + lax.axis_index("c")
    base = wid * units

    bufs = ((ids_v0, rat_v0, pe_v0, posb_v0, out_v0, mask_v0),
            (ids_v1, rat_v1, pe_v1, posb_v1, out_v1, mask_v1))

    def in_descs(ug, slot):
        ids_v, rat_v, pe_v, posb_v, _, _ = bufs[slot]
        n = ug // nb
        boff = (ug - n * nb) * _BC
        return (
            pltpu.make_async_copy(
                ids_hbm.at[pl.ds(n * bw + boff, _BC)], ids_v, in_sem),
            pltpu.make_async_copy(
                rat_hbm.at[pl.ds(n * bw + boff, _BC)], rat_v, in_sem),
            pltpu.make_async_copy(
                pe_hbm.at[pl.ds(n * _ITEM, _ITEM), pl.ds(boff, _BC)],
                pe_v, in_sem),
            pltpu.make_async_copy(
                posb_hbm.at[pl.ds(n * _D * _L, _D * _L)], posb_v, in_sem),
        )

    def out_descs(ug, slot):
        _, _, _, _, out_v, mask_v = bufs[slot]
        n = ug // nb
        boff = (ug - n * nb) * _BC
        return (
            pltpu.make_async_copy(
                out_v, out_hbm.at[pl.ds(n * _D, _D), pl.ds(boff, _BC)],
                out_sem),
            pltpu.make_async_copy(
                mask_v, mask_hbm.at[pl.ds(n * bw + boff, _BC)], out_sem),
        )

    def compute(slot):
        ids_v, rat_v, pe_v, posb_v, out_v, mask_v = bufs[slot]

        @plsc.parallel_loop(0, _BC // _L, unroll=4)
        def b16_body(k):
            c = k * _L
            ids16 = ids_v[pl.ds(c, _L)]
            rat16 = rat_v[pl.ds(c, _L)]
            validf = jnp.where(ids16 != jnp.zeros((_L,), jnp.int32), 1.0, 0.0)
            mask_v[pl.ds(c, _L)] = validf
            v8 = validf * jnp.full((_L,), 8.0, jnp.float32)
            rix = rat16[:, None]
            dn = lax.GatherDimensionNumbers(
                offset_dims=(), collapsed_slice_dims=(0,),
                start_index_map=(0,))
            for d in range(_ITEM):
                pos16 = posb_v[pl.ds(d * _L, _L)]
                pe16 = pe_v[d, pl.ds(c, _L)]
                out_v[d, pl.ds(c, _L)] = pe16 * v8 + pos16 * validf
            for e in range(_D - _ITEM):
                d = _ITEM + e
                pos16 = posb_v[pl.ds(d * _L, _L)]
                row16 = re_v[pl.ds(e * _L, _L)]
                g16 = lax.gather(
                    row16, rix, dn, (1,),
                    mode=lax.GatherScatterMode.PROMISE_IN_BOUNDS)
                out_v[d, pl.ds(c, _L)] = (g16 + pos16) * validf

    pltpu.sync_copy(re8_hbm, re_v)
    for dsc in in_descs(base, 0):
        dsc.start()

    def pair_body(p, carry):
        for slot in (0, 1):
            u = 2 * p + slot
            for dsc in in_descs(base + u, slot):
                dsc.wait()

            @pl.when(u + 1 < units)
            def _():
                for dsc in in_descs(base + u + 1, 1 - slot):
                    dsc.start()

            @pl.when(u >= 2)
            def _():
                for dsc in out_descs(base + u - 2, slot):
                    dsc.wait()

            compute(slot)
            for dsc in out_descs(base + u, slot):
                dsc.start()
        return carry

    lax.fori_loop(0, units // 2, pair_body, 0)
    for dsc in out_descs(base + units - 2, 0):
        dsc.wait()
    for dsc in out_descs(base + units - 1, 1):
        dsc.wait()


def _make_sc_call(n_rows, b):
    mesh = plsc.VectorSubcoreMesh(core_axis_name="c", subcore_axis_name="s")
    buf = [
        pltpu.VMEM((_BC,), jnp.int32),
        pltpu.VMEM((_BC,), jnp.int32),
        pltpu.VMEM((_ITEM, _BC), jnp.float32),
        pltpu.VMEM((_D * _L,), jnp.float32),
        pltpu.VMEM((_D, _BC), jnp.float32),
        pltpu.VMEM((_BC,), jnp.float32),
    ]
    return functools.partial(
        pl.kernel,
        out_type=(jax.ShapeDtypeStruct((n_rows * _D, b), jnp.float32),
                  jax.ShapeDtypeStruct((n_rows * b,), jnp.float32)),
        mesh=mesh,
        scratch_types=buf + buf + [
            pltpu.VMEM((_ITEM * _L,), jnp.float32),
            pltpu.SemaphoreType.DMA,
            pltpu.SemaphoreType.DMA,
        ],
    )(_tec_body)


def kernel(past_lengths, past_ids, past_embeddings, ratings, pos_emb,
           rating_emb):
    B, N = past_ids.shape
    D = pos_emb.shape[1]
    item = past_embeddings.shape[-1]
    scale = float(D) ** 0.5
    pe2d = jnp.transpose(past_embeddings, (1, 2, 0)).reshape(N * item, B)
    ids1d = past_ids.T.reshape(-1).astype(jnp.int32)
    rat1d = ratings.T.reshape(-1).astype(jnp.int32)
    posb = jnp.repeat(pos_emb.reshape(-1), _L)
    re8 = jnp.pad(rating_emb.T * scale,
                  ((0, 0), (0, _L - rating_emb.shape[0]))).reshape(-1)
    out2d, mask1d = _make_sc_call(N, B)(ids1d, rat1d, pe2d, posb, re8)
    out = jnp.transpose(out2d.reshape(N, D, B), (2, 0, 1))
    mask = jnp.transpose(mask1d.reshape(N, B), (1, 0))[..., None]
    return past_lengths, out, mask


# speedup vs baseline: 1.8657x; 1.8657x over previous
import functools

import jax
import jax.numpy as jnp
from jax import lax
from jax.experimental import pallas as pl
from jax.experimental.pallas import tpu as pltpu
from jax.experimental.pallas import tpu_sc as plsc

_NC = 2
_NS = 16
_NW = _NC * _NS
_L = 16
_BC = 512
_ITEM = 32
_D = 64


def _tec_body(ids_hbm, rat_hbm, pe_hbm, posb_hbm, re8_hbm,
              out_hbm, mask_hbm,
              ids_v0, rat_v0, pe_v0, posb_v0, out_v0, mask_v0,
              ids_v1, rat_v1, pe_v1, posb_v1, out_v1, mask_v1,
              re_v, in_sem, out_sem):
    bw = pe_hbm.shape[1]
    n_rows = posb_hbm.shape[0] // (_D * _L)
    nb = bw // _BC
    units = n_rows * nb // _NW
    wid = lax.axis_index("s") * _NC + lax.axis_index("c")
    base = wid * units

    bufs = ((ids_v0, rat_v0, pe_v0, posb_v0, out_v0, mask_v0),
            (ids_v1, rat_v1, pe_v1, posb_v1, out_v1, mask_v1))

    def in_descs(ug, slot):
        ids_v, rat_v, pe_v, posb_v, _, _ = bufs[slot]
        n = ug // nb
        boff = (ug - n * nb) * _BC
        return (
            pltpu.make_async_copy(
                ids_hbm.at[pl.ds(n * bw + boff, _BC)], ids_v, in_sem),
            pltpu.make_async_copy(
                rat_hbm.at[pl.ds(n * bw + boff, _BC)], rat_v, in_sem),
            pltpu.make_async_copy(
                pe_hbm.at[pl.ds(n * _ITEM, _ITEM), pl.ds(boff, _BC)],
                pe_v, in_sem),
            pltpu.make_async_copy(
                posb_hbm.at[pl.ds(n * _D * _L, _D * _L)], posb_v, in_sem),
        )

    def out_descs(ug, slot):
        _, _, _, _, out_v, mask_v = bufs[slot]
        n = ug // nb
        boff = (ug - n * nb) * _BC
        return (
            pltpu.make_async_copy(
                out_v, out_hbm.at[pl.ds(n * _D, _D), pl.ds(boff, _BC)],
                out_sem),
            pltpu.make_async_copy(
                mask_v, mask_hbm.at[pl.ds(n * bw + boff, _BC)], out_sem),
        )

    def compute(slot):
        ids_v, rat_v, pe_v, posb_v, out_v, mask_v = bufs[slot]

        @plsc.parallel_loop(0, _BC // _L, unroll=1)
        def b16_body(k):
            c = k * _L
            ids16 = ids_v[pl.ds(c, _L)]
            rat16 = rat_v[pl.ds(c, _L)]
            validf = jnp.where(ids16 != jnp.zeros((_L,), jnp.int32), 1.0, 0.0)
            mask_v[pl.ds(c, _L)] = validf
            v8 = validf * jnp.full((_L,), 8.0, jnp.float32)
            rix = rat16[:, None]
            dn = lax.GatherDimensionNumbers(
                offset_dims=(), collapsed_slice_dims=(0,),
                start_index_map=(0,))
            for d in range(_ITEM):
                pos16 = posb_v[pl.ds(d * _L, _L)]
                pe16 = pe_v[d, pl.ds(c, _L)]
                out_v[d, pl.ds(c, _L)] = pe16 * v8 + pos16 * validf
            for e in range(_D - _ITEM):
                d = _ITEM + e
                pos16 = posb_v[pl.ds(d * _L, _L)]
                row16 = re_v[pl.ds(e * _L, _L)]
                g16 = lax.gather(
                    row16, rix, dn, (1,),
                    mode=lax.GatherScatterMode.PROMISE_IN_BOUNDS)
                out_v[d, pl.ds(c, _L)] = (g16 + pos16) * validf

    pltpu.sync_copy(re8_hbm, re_v)
    for dsc in in_descs(base, 0):
        dsc.start()

    def pair_body(p, carry):
        for slot in (0, 1):
            u = 2 * p + slot
            for dsc in in_descs(base + u, slot):
                dsc.wait()

            @pl.when(u + 1 < units)
            def _():
                for dsc in in_descs(base + u + 1, 1 - slot):
                    dsc.start()

            @pl.when(u >= 2)
            def _():
                for dsc in out_descs(base + u - 2, slot):
                    dsc.wait()

            compute(slot)
            for dsc in out_descs(base + u, slot):
                dsc.start()
        return carry

    lax.fori_loop(0, units // 2, pair_body, 0)
    for dsc in out_descs(base + units - 2, 0):
        dsc.wait()
    for dsc in out_descs(base + units - 1, 1):
        dsc.wait()


def _make_sc_call(n_rows, b):
    mesh = plsc.VectorSubcoreMesh(core_axis_name="c", subcore_axis_name="s")
    buf = [
        pltpu.VMEM((_BC,), jnp.int32),
        pltpu.VMEM((_BC,), jnp.int32),
        pltpu.VMEM((_ITEM, _BC), jnp.float32),
        pltpu.VMEM((_D * _L,), jnp.float32),
        pltpu.VMEM((_D, _BC), jnp.float32),
        pltpu.VMEM((_BC,), jnp.float32),
    ]
    return functools.partial(
        pl.kernel,
        out_type=(jax.ShapeDtypeStruct((n_rows * _D, b), jnp.float32),
                  jax.ShapeDtypeStruct((n_rows * b,), jnp.float32)),
        mesh=mesh,
        scratch_types=buf + buf + [
            pltpu.VMEM((_ITEM * _L,), jnp.float32),
            pltpu.SemaphoreType.DMA,
            pltpu.SemaphoreType.DMA,
        ],
    )(_tec_body)


def kernel(past_lengths, past_ids, past_embeddings, ratings, pos_emb,
           rating_emb):
    B, N = past_ids.shape
    D = pos_emb.shape[1]
    item = past_embeddings.shape[-1]
    scale = float(D) ** 0.5
    pe2d = jnp.transpose(past_embeddings, (1, 2, 0)).reshape(N * item, B)
    ids1d = past_ids.T.reshape(-1).astype(jnp.int32)
    rat1d = ratings.T.reshape(-1).astype(jnp.int32)
    posb = jnp.repeat(pos_emb.reshape(-1), _L)
    re8 = jnp.pad(rating_emb.T * scale,
                  ((0, 0), (0, _L - rating_emb.shape[0]))).reshape(-1)
    out2d, mask1d = _make_sc_call(N, B)(ids1d, rat1d, pe2d, posb, re8)
    out = jnp.transpose(out2d.reshape(N, D, B), (2, 0, 1))
    mask = jnp.transpose(mask1d.reshape(N, B), (1, 0))[..., None]
    return past_lengths, out, mask
